# baseline (device time: 333455 ns/iter reference)
import jax
import jax.numpy as jnp
from jax import lax
from jax.experimental import pallas as pl
from jax.experimental.pallas import tpu as pltpu

N_DEV = 4
NB = 1024
N_BLOCKS = 8192 // NB
NBC = NB // 2


def kernel(x, w_mat, scale_x, scale_w):
    m_total, k_loc = x.shape
    _, n = w_mat.shape
    m_chunk = m_total // N_DEV
    n_blocks = n // NB
    half = n_blocks // 2
    n_msgs = (N_DEV - 1) * n_blocks

    x8 = x.astype(jnp.float8_e4m3fn)
    w8 = w_mat.astype(jnp.float8_e5m2)

    def body(x_ref, w_ref, sx_ref, sw_ref, out_ref,
             send_buf, recv_buf, out_stage,
             send_sems, recv_sems, copy_sems, credit_cw, credit_ccw):
        my = lax.axis_index("i")
        left = lax.rem(my - 1 + N_DEV, N_DEV)
        right = lax.rem(my + 1, N_DEV)

        barrier = pltpu.get_barrier_semaphore()
        for nbr in (left, right):
            pl.semaphore_signal(barrier, inc=1, device_id=(nbr,),
                                device_id_type=pl.DeviceIdType.MESH)
        pl.semaphore_wait(barrier, 2)

        s_scale = sx_ref[0] * sw_ref[0]

        def partial_tile(c, b, h):
            xt = x_ref[pl.ds(c * m_chunk, m_chunk), :]
            wt = w_ref[:, pl.ds(b * NB + h * NBC, NBC)]
            return lax.dot_general(xt, wt, (((1,), (0,)), ((), ())),
                                   preferred_element_type=jnp.float32)

        def send_rdma(slot, b, dest):
            return pltpu.make_async_remote_copy(
                src_ref=send_buf.at[slot],
                dst_ref=recv_buf.at[b],
                send_sem=send_sems.at[slot],
                recv_sem=recv_sems.at[b],
                device_id=(dest,),
                device_id_type=pl.DeviceIdType.MESH,
            )

        def wait_recv(b):
            send_rdma(0, b, my).wait_recv()

        def wait_send(slot):
            send_rdma(slot, 0, my).wait_send()

        def ring_msg(k, carry):
            s = lax.div(k, n_blocks)
            j = lax.rem(k, n_blocks)
            b = lax.rem(j, 2) * half + lax.div(j, 2)
            cw = b < half
            dest = jnp.where(cw, right, left)
            upstream = jnp.where(cw, left, right)
            slot = lax.rem(b, 2) + jnp.where(cw, 0, 2)
            c = jnp.where(
                cw,
                lax.rem(my - 1 - s + 2 * N_DEV, N_DEV),
                lax.rem(my + 1 + s, N_DEV),
            )

            @pl.when(s > 0)
            def _():
                wait_recv(b)

            @pl.when((s > 0) | (lax.rem(b, half) >= 2))
            def _():
                wait_send(slot)

            for h in range(2):
                acc = partial_tile(c, b, h)

                @pl.when(s == 0)
                def _(acc=acc, h=h):
                    send_buf[slot, :, h * NBC:(h + 1) * NBC] = (
                        acc.astype(jnp.bfloat16))

                @pl.when(s > 0)
                def _(acc=acc, h=h):
                    send_buf[slot, :, h * NBC:(h + 1) * NBC] = (
                        acc + recv_buf[b, :, h * NBC:(h + 1) * NBC].astype(
                            jnp.float32)).astype(jnp.bfloat16)

            @pl.when((s > 0) & cw)
            def _():
                pl.semaphore_signal(credit_cw, inc=1, device_id=(left,),
                                    device_id_type=pl.DeviceIdType.MESH)
                pl.semaphore_wait(credit_cw, 1)

            @pl.when((s > 0) & jnp.logical_not(cw))
            def _():
                pl.semaphore_signal(credit_ccw, inc=1, device_id=(right,),
                                    device_id_type=pl.DeviceIdType.MESH)
                pl.semaphore_wait(credit_ccw, 1)

            send_rdma(slot, b, dest).start()
            return carry

        lax.fori_loop(0, n_msgs, ring_msg, 0)

        def final_msg(j, carry):
            b = lax.rem(j, 2) * half + lax.div(j, 2)
            wait_recv(b)
            st = lax.rem(j, 2)

            @pl.when(j >= 2)
            def _():
                pltpu.make_async_copy(
                    out_stage.at[st],
                    out_ref.at[:, pl.ds((b - 1) * NB, NB)],
                    copy_sems.at[st]).wait()

            for h in range(2):
                acc = partial_tile(my, b, h) + recv_buf[
                    b, :, h * NBC:(h + 1) * NBC].astype(jnp.float32)
                y = acc * s_scale
                out_stage[st, :, h * NBC:(h + 1) * NBC] = (
                    y * jax.nn.sigmoid(y))

            pltpu.make_async_copy(
                out_stage.at[st], out_ref.at[:, pl.ds(b * NB, NB)],
                copy_sems.at[st]).start()
            return carry

        lax.fori_loop(0, n_blocks, final_msg, 0)

        for st, b in ((0, half - 1), (1, n_blocks - 1)):
            pltpu.make_async_copy(
                out_stage.at[st], out_ref.at[:, pl.ds(b * NB, NB)],
                copy_sems.at[st]).wait()
        for slot in range(4):
            wait_send(slot)

    return pl.pallas_call(
        body,
        out_shape=jax.ShapeDtypeStruct((m_chunk, n), jnp.float32),
        in_specs=[
            pl.BlockSpec(memory_space=pltpu.VMEM),
            pl.BlockSpec(memory_space=pltpu.VMEM),
            pl.BlockSpec(memory_space=pltpu.SMEM),
            pl.BlockSpec(memory_space=pltpu.SMEM),
        ],
        out_specs=pl.BlockSpec(memory_space=pl.ANY),
        scratch_shapes=[
            pltpu.VMEM((4, m_chunk, NB), jnp.bfloat16),
            pltpu.VMEM((N_BLOCKS, m_chunk, NB), jnp.bfloat16),
            pltpu.VMEM((2, m_chunk, NB), jnp.float32),
            pltpu.SemaphoreType.DMA((4,)),
            pltpu.SemaphoreType.DMA((N_BLOCKS,)),
            pltpu.SemaphoreType.DMA((2,)),
            pltpu.SemaphoreType.REGULAR,
            pltpu.SemaphoreType.REGULAR,
        ],
        compiler_params=pltpu.CompilerParams(
            collective_id=0, vmem_limit_bytes=58 * 1024 * 1024),
    )(x8, w8, scale_x, scale_w)


# device time: 318060 ns/iter; 1.0484x vs baseline; 1.0484x over previous
import jax
import jax.numpy as jnp
from jax import lax
from jax.experimental import pallas as pl
from jax.experimental.pallas import tpu as pltpu

N_DEV = 4
NB = 1024
N_BLOCKS = 8192 // NB
NBC = NB // 2


def kernel(x, w_mat, scale_x, scale_w):
    m_total, k_loc = x.shape
    _, n = w_mat.shape
    m_chunk = m_total // N_DEV
    n_blocks = n // NB
    half = n_blocks // 2
    n_msgs = (N_DEV - 1) * n_blocks

    x8 = x.astype(jnp.float8_e4m3fn)

    def body(x_ref, w_ref, sx_ref, sw_ref, out_ref,
             send_buf, recv_buf, out_stage, w8_buf,
             send_sems, recv_sems, copy_sems, credit_cw, credit_ccw):
        my = lax.axis_index("i")
        left = lax.rem(my - 1 + N_DEV, N_DEV)
        right = lax.rem(my + 1, N_DEV)

        def w_dma(b, slot):
            return pltpu.make_async_copy(
                w_ref.at[:, pl.ds(b * NB, NB)], out_stage.at[slot],
                copy_sems.at[slot])

        w_dma(0, 0).start()
        w_dma(half, 1).start()

        barrier = pltpu.get_barrier_semaphore()
        for nbr in (left, right):
            pl.semaphore_signal(barrier, inc=1, device_id=(nbr,),
                                device_id_type=pl.DeviceIdType.MESH)
        pl.semaphore_wait(barrier, 2)

        s_scale = sx_ref[0] * sw_ref[0]

        def partial_tile(c, b, h):
            xt = x_ref[pl.ds(c * m_chunk, m_chunk), :]
            wt = w8_buf[:, pl.ds(b * NB + h * NBC, NBC)]
            return lax.dot_general(xt, wt, (((1,), (0,)), ((), ())),
                                   preferred_element_type=jnp.float32)

        def send_rdma(slot, b, dest):
            return pltpu.make_async_remote_copy(
                src_ref=send_buf.at[slot],
                dst_ref=recv_buf.at[b],
                send_sem=send_sems.at[slot],
                recv_sem=recv_sems.at[b],
                device_id=(dest,),
                device_id_type=pl.DeviceIdType.MESH,
            )

        def wait_recv(b):
            send_rdma(0, b, my).wait_recv()

        def wait_send(slot):
            send_rdma(slot, 0, my).wait_send()

        def ring_msg(k, carry):
            s = lax.div(k, n_blocks)
            j = lax.rem(k, n_blocks)
            b = lax.rem(j, 2) * half + lax.div(j, 2)
            cw = b < half
            dest = jnp.where(cw, right, left)
            upstream = jnp.where(cw, left, right)
            slot = lax.rem(b, 2) + jnp.where(cw, 0, 2)
            c = jnp.where(
                cw,
                lax.rem(my - 1 - s + 2 * N_DEV, N_DEV),
                lax.rem(my + 1 + s, N_DEV),
            )

            @pl.when(s == 0)
            def _():
                slot_w = lax.rem(j, 2)
                w_dma(b, slot_w).wait()
                w8_buf[:, pl.ds(b * NB, NB)] = (
                    out_stage[slot_w].astype(jnp.float8_e5m2))

                @pl.when(j < n_blocks - 2)
                def _():
                    w_dma(b + 1, slot_w).start()

            @pl.when(s > 0)
            def _():
                wait_recv(b)

            @pl.when((s > 0) | (lax.rem(b, half) >= 2))
            def _():
                wait_send(slot)

            for h in range(2):
                acc = partial_tile(c, b, h)

                @pl.when(s == 0)
                def _(acc=acc, h=h):
                    send_buf[slot, :, h * NBC:(h + 1) * NBC] = (
                        acc.astype(jnp.bfloat16))

                @pl.when(s > 0)
                def _(acc=acc, h=h):
                    send_buf[slot, :, h * NBC:(h + 1) * NBC] = (
                        acc + recv_buf[b, :, h * NBC:(h + 1) * NBC].astype(
                            jnp.float32)).astype(jnp.bfloat16)

            @pl.when((s > 0) & cw)
            def _():
                pl.semaphore_signal(credit_cw, inc=1, device_id=(left,),
                                    device_id_type=pl.DeviceIdType.MESH)
                pl.semaphore_wait(credit_cw, 1)

            @pl.when((s > 0) & jnp.logical_not(cw))
            def _():
                pl.semaphore_signal(credit_ccw, inc=1, device_id=(right,),
                                    device_id_type=pl.DeviceIdType.MESH)
                pl.semaphore_wait(credit_ccw, 1)

            send_rdma(slot, b, dest).start()
            return carry

        lax.fori_loop(0, n_msgs, ring_msg, 0)

        def final_msg(j, carry):
            b = lax.rem(j, 2) * half + lax.div(j, 2)
            wait_recv(b)
            st = lax.rem(j, 2)

            @pl.when(j >= 2)
            def _():
                pltpu.make_async_copy(
                    out_stage.at[st],
                    out_ref.at[:, pl.ds((b - 1) * NB, NB)],
                    copy_sems.at[st]).wait()

            for h in range(2):
                acc = partial_tile(my, b, h) + recv_buf[
                    b, :, h * NBC:(h + 1) * NBC].astype(jnp.float32)
                y = acc * s_scale
                out_stage[st, :, h * NBC:(h + 1) * NBC] = (
                    y * jax.nn.sigmoid(y))

            pltpu.make_async_copy(
                out_stage.at[st], out_ref.at[:, pl.ds(b * NB, NB)],
                copy_sems.at[st]).start()
            return carry

        lax.fori_loop(0, n_blocks, final_msg, 0)

        for st, b in ((0, half - 1), (1, n_blocks - 1)):
            pltpu.make_async_copy(
                out_stage.at[st], out_ref.at[:, pl.ds(b * NB, NB)],
                copy_sems.at[st]).wait()
        for slot in range(4):
            wait_send(slot)

    return pl.pallas_call(
        body,
        out_shape=jax.ShapeDtypeStruct((m_chunk, n), jnp.float32),
        in_specs=[
            pl.BlockSpec(memory_space=pltpu.VMEM),
            pl.BlockSpec(memory_space=pl.ANY),
            pl.BlockSpec(memory_space=pltpu.SMEM),
            pl.BlockSpec(memory_space=pltpu.SMEM),
        ],
        out_specs=pl.BlockSpec(memory_space=pl.ANY),
        scratch_shapes=[
            pltpu.VMEM((4, m_chunk, NB), jnp.bfloat16),
            pltpu.VMEM((N_BLOCKS, m_chunk, NB), jnp.bfloat16),
            pltpu.VMEM((2, m_chunk, NB), jnp.float32),
            pltpu.VMEM((k_loc, n), jnp.float8_e5m2),
            pltpu.SemaphoreType.DMA((4,)),
            pltpu.SemaphoreType.DMA((N_BLOCKS,)),
            pltpu.SemaphoreType.DMA((2,)),
            pltpu.SemaphoreType.REGULAR,
            pltpu.SemaphoreType.REGULAR,
        ],
        compiler_params=pltpu.CompilerParams(
            collective_id=0, vmem_limit_bytes=58 * 1024 * 1024),
    )(x8, w_mat, scale_x, scale_w)


# device time: 309783 ns/iter; 1.0764x vs baseline; 1.0267x over previous
import jax
import jax.numpy as jnp
from jax import lax
from jax.experimental import pallas as pl
from jax.experimental.pallas import tpu as pltpu

N_DEV = 4
NB = 1024
N_BLOCKS = 8192 // NB
NBC = NB // 2


def kernel(x, w_mat, scale_x, scale_w):
    m_total, k_loc = x.shape
    _, n = w_mat.shape
    m_chunk = m_total // N_DEV
    n_blocks = n // NB
    half = n_blocks // 2
    n_msgs = (N_DEV - 1) * n_blocks


    def body(x_ref, w_ref, sx_ref, sw_ref, out_ref,
             send_buf, recv_buf, out_stage, w8_buf, x8_buf,
             send_sems, recv_sems, copy_sems, credit_cw, credit_ccw):
        my = lax.axis_index("i")
        left = lax.rem(my - 1 + N_DEV, N_DEV)
        right = lax.rem(my + 1, N_DEV)

        def w_dma(b, slot):
            return pltpu.make_async_copy(
                w_ref.at[:, pl.ds(b * NB, NB)], out_stage.at[slot],
                copy_sems.at[slot])

        def x_dma(c, slot):
            return pltpu.make_async_copy(
                x_ref.at[pl.ds(c * m_chunk, m_chunk), :], out_stage.at[slot],
                copy_sems.at[slot])

        def x_convert(c, slot):
            x8_buf[pl.ds(c * m_chunk, m_chunk), :] = (
                out_stage[slot].astype(jnp.float8_e4m3fn))

        c_cw0 = lax.rem(my - 1 + N_DEV, N_DEV)
        c_ccw0 = lax.rem(my + 1, N_DEV)
        x_dma(c_cw0, 0).start()
        x_dma(c_ccw0, 1).start()
        x_dma(c_cw0, 0).wait()
        x_convert(c_cw0, 0)
        w_dma(0, 0).start()
        x_dma(c_ccw0, 1).wait()
        x_convert(c_ccw0, 1)
        w_dma(half, 1).start()

        barrier = pltpu.get_barrier_semaphore()
        for nbr in (left, right):
            pl.semaphore_signal(barrier, inc=1, device_id=(nbr,),
                                device_id_type=pl.DeviceIdType.MESH)
        pl.semaphore_wait(barrier, 2)

        s_scale = sx_ref[0] * sw_ref[0]

        def partial_tile(c, b, h):
            xt = x8_buf[pl.ds(c * m_chunk, m_chunk), :]
            wt = w8_buf[:, pl.ds(b * NB + h * NBC, NBC)]
            return lax.dot_general(xt, wt, (((1,), (0,)), ((), ())),
                                   preferred_element_type=jnp.float32)

        def send_rdma(slot, b, dest):
            return pltpu.make_async_remote_copy(
                src_ref=send_buf.at[slot],
                dst_ref=recv_buf.at[b],
                send_sem=send_sems.at[slot],
                recv_sem=recv_sems.at[b],
                device_id=(dest,),
                device_id_type=pl.DeviceIdType.MESH,
            )

        def wait_recv(b):
            send_rdma(0, b, my).wait_recv()

        def wait_send(slot):
            send_rdma(slot, 0, my).wait_send()

        def ring_msg(k, carry):
            s = lax.div(k, n_blocks)
            j = lax.rem(k, n_blocks)
            b = lax.rem(j, 2) * half + lax.div(j, 2)
            cw = b < half
            dest = jnp.where(cw, right, left)
            upstream = jnp.where(cw, left, right)
            slot = lax.rem(b, 2) + jnp.where(cw, 0, 2)
            c = jnp.where(
                cw,
                lax.rem(my - 1 - s + 2 * N_DEV, N_DEV),
                lax.rem(my + 1 + s, N_DEV),
            )

            @pl.when(s == 0)
            def _():
                slot_w = lax.rem(j, 2)
                w_dma(b, slot_w).wait()
                w8_buf[:, pl.ds(b * NB, NB)] = (
                    out_stage[slot_w].astype(jnp.float8_e5m2))

                @pl.when(j < n_blocks - 2)
                def _():
                    w_dma(b + 1, slot_w).start()

                @pl.when(j == n_blocks - 1)
                def _():
                    x_dma(lax.rem(my + 2, N_DEV), 0).start()

            @pl.when((s == 1) & (j == 0))
            def _():
                c2 = lax.rem(my + 2, N_DEV)
                x_dma(c2, 0).wait()
                x_convert(c2, 0)
                x_dma(my, 1).start()

            @pl.when((s == 1) & (j == 1))
            def _():
                x_dma(my, 1).wait()
                x_convert(my, 1)

            @pl.when(s > 0)
            def _():
                wait_recv(b)

            @pl.when((s > 0) | (lax.rem(b, half) >= 2))
            def _():
                wait_send(slot)

            for h in range(2):
                acc = partial_tile(c, b, h)

                @pl.when(s == 0)
                def _(acc=acc, h=h):
                    send_buf[slot, :, h * NBC:(h + 1) * NBC] = (
                        acc.astype(jnp.bfloat16))

                @pl.when(s > 0)
                def _(acc=acc, h=h):
                    send_buf[slot, :, h * NBC:(h + 1) * NBC] = (
                        acc + recv_buf[b, :, h * NBC:(h + 1) * NBC].astype(
                            jnp.float32)).astype(jnp.bfloat16)

            @pl.when((s > 0) & cw)
            def _():
                pl.semaphore_signal(credit_cw, inc=1, device_id=(left,),
                                    device_id_type=pl.DeviceIdType.MESH)
                pl.semaphore_wait(credit_cw, 1)

            @pl.when((s > 0) & jnp.logical_not(cw))
            def _():
                pl.semaphore_signal(credit_ccw, inc=1, device_id=(right,),
                                    device_id_type=pl.DeviceIdType.MESH)
                pl.semaphore_wait(credit_ccw, 1)

            send_rdma(slot, b, dest).start()
            return carry

        lax.fori_loop(0, n_msgs, ring_msg, 0)

        def final_msg(j, carry):
            b = lax.rem(j, 2) * half + lax.div(j, 2)
            wait_recv(b)
            st = lax.rem(j, 2)

            @pl.when(j >= 2)
            def _():
                pltpu.make_async_copy(
                    out_stage.at[st],
                    out_ref.at[:, pl.ds((b - 1) * NB, NB)],
                    copy_sems.at[st]).wait()

            for h in range(2):
                acc = partial_tile(my, b, h) + recv_buf[
                    b, :, h * NBC:(h + 1) * NBC].astype(jnp.float32)
                y = acc * s_scale
                out_stage[st, :, h * NBC:(h + 1) * NBC] = (
                    y * jax.nn.sigmoid(y))

            pltpu.make_async_copy(
                out_stage.at[st], out_ref.at[:, pl.ds(b * NB, NB)],
                copy_sems.at[st]).start()
            return carry

        lax.fori_loop(0, n_blocks, final_msg, 0)

        for st, b in ((0, half - 1), (1, n_blocks - 1)):
            pltpu.make_async_copy(
                out_stage.at[st], out_ref.at[:, pl.ds(b * NB, NB)],
                copy_sems.at[st]).wait()
        for slot in range(4):
            wait_send(slot)

    return pl.pallas_call(
        body,
        out_shape=jax.ShapeDtypeStruct((m_chunk, n), jnp.float32),
        in_specs=[
            pl.BlockSpec(memory_space=pl.ANY),
            pl.BlockSpec(memory_space=pl.ANY),
            pl.BlockSpec(memory_space=pltpu.SMEM),
            pl.BlockSpec(memory_space=pltpu.SMEM),
        ],
        out_specs=pl.BlockSpec(memory_space=pl.ANY),
        scratch_shapes=[
            pltpu.VMEM((4, m_chunk, NB), jnp.bfloat16),
            pltpu.VMEM((N_BLOCKS, m_chunk, NB), jnp.bfloat16),
            pltpu.VMEM((2, m_chunk, NB), jnp.float32),
            pltpu.VMEM((k_loc, n), jnp.float8_e5m2),
            pltpu.VMEM((m_total, k_loc), jnp.float8_e4m3fn),
            pltpu.SemaphoreType.DMA((4,)),
            pltpu.SemaphoreType.DMA((N_BLOCKS,)),
            pltpu.SemaphoreType.DMA((2,)),
            pltpu.SemaphoreType.REGULAR,
            pltpu.SemaphoreType.REGULAR,
        ],
        compiler_params=pltpu.CompilerParams(
            collective_id=0, vmem_limit_bytes=58 * 1024 * 1024),
    )(x, w_mat, scale_x, scale_w)
